# baseline (speedup 1.0000x reference)
import functools

import jax
import jax.numpy as jnp
from jax import lax
from jax.experimental import pallas as pl
from jax.experimental.pallas import tpu as pltpu
from jax.experimental.pallas import tpu_sc as plsc

_NC = 2
_NS = 16
_L = 16
_NW = _NC * _NS

_B = 16384
_D = 32
_BPW = _B // _NW
_CHUNK = 128
_NCH = _BPW // _CHUNK


def _mf_body(u_hbm, v_hbm, ue_hbm, ve_hbm, out_hbm,
             ui_v, vi_v, uq_v, vq_v, ue_b, ve_b, o_v, semu, semv):
    wid = lax.axis_index("s") * _NC + lax.axis_index("c")
    base = wid * _BPW

    pltpu.sync_copy(u_hbm.at[pl.ds(base, _BPW)], ui_v)
    pltpu.sync_copy(v_hbm.at[pl.ds(base, _BPW)], vi_v)

    def quarter(i, carry):
        sl = pl.ds(i * _L, _L)
        uq_v[sl] = lax.shift_right_logical(ui_v[sl], 2)
        vq_v[sl] = lax.shift_right_logical(vi_v[sl], 2)
        return carry

    lax.fori_loop(0, _BPW // _L, quarter, 0)

    iota = lax.iota(jnp.int32, _L)

    for c in range(_NCH):
        csl = pl.ds(c * _CHUNK, _CHUNK)
        cu = pltpu.async_copy(ue_hbm.at[uq_v.at[csl]], ue_b, semu)
        cv = pltpu.async_copy(ve_hbm.at[vq_v.at[csl]], ve_b, semv)
        cu.wait()
        cv.wait()

        def group_body(g, carry, c=c):
            row0 = g * _L
            rows = row0 + iota
            ucols = (ui_v[pl.ds(c * _CHUNK + row0, _L)] & 3) * _D
            vcols = (vi_v[pl.ds(c * _CHUNK + row0, _L)] & 3) * _D

            def d_body(d, acc):
                a = plsc.load_gather(ue_b, [rows, ucols + d])
                b = plsc.load_gather(ve_b, [rows, vcols + d])
                return acc + a * b

            acc = lax.fori_loop(0, _D, d_body, jnp.zeros((_L,), jnp.float32))
            o_v[pl.ds(c * _CHUNK + row0, _L)] = (
                4.0 / (1.0 + jnp.exp(-acc)) + 1.0
            )
            return carry

        lax.fori_loop(0, _CHUNK // _L, group_body, 0)

    pltpu.sync_copy(o_v, out_hbm.at[pl.ds(base, _BPW)])


def kernel(u, v, user_emb, movie_emb):
    ue2 = user_emb.reshape(-1, 4 * _D)
    ve2 = movie_emb.reshape(-1, 4 * _D)
    mesh = plsc.VectorSubcoreMesh(core_axis_name="c", subcore_axis_name="s")
    run = functools.partial(
        pl.kernel,
        out_type=jax.ShapeDtypeStruct((_B,), jnp.float32),
        mesh=mesh,
        compiler_params=pltpu.CompilerParams(needs_layout_passes=False),
        scratch_types=[
            pltpu.VMEM((_BPW,), jnp.int32),
            pltpu.VMEM((_BPW,), jnp.int32),
            pltpu.VMEM((_BPW,), jnp.int32),
            pltpu.VMEM((_BPW,), jnp.int32),
            pltpu.VMEM((_CHUNK, 4 * _D), jnp.float32),
            pltpu.VMEM((_CHUNK, 4 * _D), jnp.float32),
            pltpu.VMEM((_BPW,), jnp.float32),
            pltpu.SemaphoreType.DMA,
            pltpu.SemaphoreType.DMA,
        ],
    )(_mf_body)
    return run(u, v, ue2, ve2)


# speedup vs baseline: 3.5672x; 3.5672x over previous
import functools

import jax
import jax.numpy as jnp
from jax import lax
from jax.experimental import pallas as pl
from jax.experimental.pallas import tpu as pltpu
from jax.experimental.pallas import tpu_sc as plsc

_NC = 2
_NS = 16
_L = 16
_NW = _NC * _NS

_B = 16384
_D = 32
_BPW = _B // _NW
_NB = 4


def _mf_body(u_hbm, v_hbm, ueT_hbm, veT_hbm, out_hbm,
             ui_v, vi_v, c32, du0, du1, dv0, dv1, o_v,
             semu0, semu1, semv0, semv1):
    wid = lax.axis_index("s") * _NC + lax.axis_index("c")
    base = wid * _BPW

    iota = lax.iota(jnp.int32, _L)
    c32[pl.ds(0, _L)] = iota
    c32[pl.ds(_L, _L)] = iota + _L

    pltpu.sync_copy(u_hbm.at[pl.ds(base, _BPW)], ui_v)
    pltpu.sync_copy(v_hbm.at[pl.ds(base, _BPW)], vi_v)

    dub = (du0, du1)
    dvb = (dv0, dv1)
    semub = (semu0, semu1)
    semvb = (semv0, semv1)

    def fire(u16, v16, b):
        p = b & 1
        cps = []
        for j in range(_NB):
            ru = u16[b * _NB + j]
            rv = v16[b * _NB + j]
            j0u = pl.multiple_of(ru & -128, 128)
            j0v = pl.multiple_of(rv & -128, 128)
            cps.append(pltpu.async_copy(
                ueT_hbm.at[c32, pl.ds(j0u, 128)], dub[p].at[j], semub[p]))
            cps.append(pltpu.async_copy(
                veT_hbm.at[c32, pl.ds(j0v, 128)], dvb[p].at[j], semvb[p]))
        return cps

    def group_body(g, carry):
        gbase = pl.multiple_of(g * _L, _L)
        u16 = ui_v[pl.ds(gbase, _L)]
        v16 = vi_v[pl.ds(gbase, _L)]

        pend = fire(u16, v16, 0)
        for b in range(_L // _NB):
            nxt = fire(u16, v16, b + 1) if b + 1 < _L // _NB else []
            for cp in pend:
                cp.wait()
            pend = nxt
            p = b & 1
            for j in range(_NB):
                ru = u16[b * _NB + j]
                rv = v16[b * _NB + j]
                rmu = jnp.zeros((_L,), jnp.int32) + (ru & 127)
                rmv = jnp.zeros((_L,), jnp.int32) + (rv & 127)
                jj = jnp.zeros((_L,), jnp.int32) + j
                a0 = plsc.load_gather(dub[p], [jj, iota, rmu])
                a1 = plsc.load_gather(dub[p], [jj, iota + _L, rmu])
                b0 = plsc.load_gather(dvb[p], [jj, iota, rmv])
                b1 = plsc.load_gather(dvb[p], [jj, iota + _L, rmv])
                e = a0 * b0 + a1 * b1
                dot = lax.reduce_sum_p.bind(e, axes=(0,))
                val = jnp.zeros((_L,), jnp.float32) + dot
                pos = jnp.zeros((_L,), jnp.int32) + (gbase + b * _NB + j)
                plsc.store_scatter(o_v, [pos], val, mask=iota == 0)
        return carry

    lax.fori_loop(0, _BPW // _L, group_body, 0)

    def sig(i, carry):
        sl = pl.ds(i * _L, _L)
        o_v[sl] = 4.0 / (1.0 + jnp.exp(-o_v[sl])) + 1.0
        return carry

    lax.fori_loop(0, _BPW // _L, sig, 0)

    pltpu.sync_copy(o_v, out_hbm.at[pl.ds(base, _BPW)])


def kernel(u, v, user_emb, movie_emb):
    ueT = user_emb.T
    veT = movie_emb.T
    mesh = plsc.VectorSubcoreMesh(core_axis_name="c", subcore_axis_name="s")
    run = functools.partial(
        pl.kernel,
        out_type=jax.ShapeDtypeStruct((_B,), jnp.float32),
        mesh=mesh,
        compiler_params=pltpu.CompilerParams(
            needs_layout_passes=False, disable_bounds_checks=True
        ),
        scratch_types=[
            pltpu.VMEM((_BPW,), jnp.int32),
            pltpu.VMEM((_BPW,), jnp.int32),
            pltpu.VMEM((2 * _L,), jnp.int32),
            pltpu.VMEM((_NB, _D, 128), jnp.float32),
            pltpu.VMEM((_NB, _D, 128), jnp.float32),
            pltpu.VMEM((_NB, _D, 128), jnp.float32),
            pltpu.VMEM((_NB, _D, 128), jnp.float32),
            pltpu.VMEM((_BPW,), jnp.float32),
            pltpu.SemaphoreType.DMA,
            pltpu.SemaphoreType.DMA,
            pltpu.SemaphoreType.DMA,
            pltpu.SemaphoreType.DMA,
        ],
    )(_mf_body)
    return run(u, v, ueT, veT)


# speedup vs baseline: 3.9393x; 1.1043x over previous
import functools

import jax
import jax.numpy as jnp
from jax import lax
from jax.experimental import pallas as pl
from jax.experimental.pallas import tpu as pltpu
from jax.experimental.pallas import tpu_sc as plsc

_NC = 2
_NS = 16
_L = 16
_NW = _NC * _NS

_B = 16384
_D = 32
_BPW = _B // _NW
_NB = 4


def _mf_body(u_hbm, v_hbm, ueT_hbm, veT_hbm, out_hbm,
             ui_v, vi_v, c32, du0, du1, dv0, dv1, o_v,
             semu0, semu1, semv0, semv1):
    wid = lax.axis_index("s") * _NC + lax.axis_index("c")
    base = wid * _BPW

    iota = lax.iota(jnp.int32, _L)
    c32[pl.ds(0, _L)] = iota
    c32[pl.ds(_L, _L)] = iota + _L

    pltpu.sync_copy(u_hbm.at[pl.ds(base, _BPW)], ui_v)
    pltpu.sync_copy(v_hbm.at[pl.ds(base, _BPW)], vi_v)

    dub = (du0, du1)
    dvb = (dv0, dv1)
    semub = (semu0, semu1)
    semvb = (semv0, semv1)

    def fire(u16, v16, b):
        p = b & 1
        cps = []
        for j in range(_NB):
            ru = u16[b * _NB + j]
            rv = v16[b * _NB + j]
            j0u = pl.multiple_of(ru & -128, 128)
            j0v = pl.multiple_of(rv & -128, 128)
            cps.append(pltpu.async_copy(
                ueT_hbm.at[c32, pl.ds(j0u, 128)], dub[p].at[j], semub[p]))
            cps.append(pltpu.async_copy(
                veT_hbm.at[c32, pl.ds(j0v, 128)], dvb[p].at[j], semvb[p]))
        return cps

    def wait_batch(p):
        for j in range(_NB):
            pltpu.make_async_copy(
                ueT_hbm.at[c32, pl.ds(0, 128)], dub[p].at[j], semub[p]
            ).wait()
            pltpu.make_async_copy(
                veT_hbm.at[c32, pl.ds(0, 128)], dvb[p].at[j], semvb[p]
            ).wait()

    def process(u16, v16, b, gbase):
        p = b & 1
        for j in range(_NB):
            ru = u16[b * _NB + j]
            rv = v16[b * _NB + j]
            rmu = jnp.zeros((_L,), jnp.int32) + (ru & 127)
            rmv = jnp.zeros((_L,), jnp.int32) + (rv & 127)
            jj = jnp.zeros((_L,), jnp.int32) + j
            a0 = plsc.load_gather(dub[p], [jj, iota, rmu])
            a1 = plsc.load_gather(dub[p], [jj, iota + _L, rmu])
            b0 = plsc.load_gather(dvb[p], [jj, iota, rmv])
            b1 = plsc.load_gather(dvb[p], [jj, iota + _L, rmv])
            e = a0 * b0 + a1 * b1
            dot = lax.reduce_sum_p.bind(e, axes=(0,))
            val = jnp.zeros((_L,), jnp.float32) + dot
            pos = jnp.zeros((_L,), jnp.int32) + (gbase + b * _NB + j)
            plsc.store_scatter(o_v, [pos], val, mask=iota == 0)

    _NBG = _L // _NB

    u16_0 = ui_v[pl.ds(0, _L)]
    v16_0 = vi_v[pl.ds(0, _L)]
    fire(u16_0, v16_0, 0)

    def group_body(g, carry):
        gbase = pl.multiple_of(g * _L, _L)
        u16 = ui_v[pl.ds(gbase, _L)]
        v16 = vi_v[pl.ds(gbase, _L)]
        nbase = pl.multiple_of((g + 1) * _L % _BPW, _L)
        u16n = ui_v[pl.ds(nbase, _L)]
        v16n = vi_v[pl.ds(nbase, _L)]

        for b in range(_NBG):
            if b + 1 < _NBG:
                fire(u16, v16, b + 1)
            else:
                fire(u16n, v16n, 0)
            wait_batch(b & 1)
            process(u16, v16, b, gbase)
        return carry

    lax.fori_loop(0, _BPW // _L, group_body, 0)
    wait_batch(0)

    def sig(i, carry):
        sl = pl.ds(i * _L, _L)
        o_v[sl] = 4.0 / (1.0 + jnp.exp(-o_v[sl])) + 1.0
        return carry

    lax.fori_loop(0, _BPW // _L, sig, 0)

    pltpu.sync_copy(o_v, out_hbm.at[pl.ds(base, _BPW)])


def kernel(u, v, user_emb, movie_emb):
    ueT = user_emb.T
    veT = movie_emb.T
    mesh = plsc.VectorSubcoreMesh(core_axis_name="c", subcore_axis_name="s")
    run = functools.partial(
        pl.kernel,
        out_type=jax.ShapeDtypeStruct((_B,), jnp.float32),
        mesh=mesh,
        compiler_params=pltpu.CompilerParams(
            needs_layout_passes=False, disable_bounds_checks=True
        ),
        scratch_types=[
            pltpu.VMEM((_BPW,), jnp.int32),
            pltpu.VMEM((_BPW,), jnp.int32),
            pltpu.VMEM((2 * _L,), jnp.int32),
            pltpu.VMEM((_NB, _D, 128), jnp.float32),
            pltpu.VMEM((_NB, _D, 128), jnp.float32),
            pltpu.VMEM((_NB, _D, 128), jnp.float32),
            pltpu.VMEM((_NB, _D, 128), jnp.float32),
            pltpu.VMEM((_BPW,), jnp.float32),
            pltpu.SemaphoreType.DMA,
            pltpu.SemaphoreType.DMA,
            pltpu.SemaphoreType.DMA,
            pltpu.SemaphoreType.DMA,
        ],
    )(_mf_body)
    return run(u, v, ueT, veT)


# speedup vs baseline: 3.9543x; 1.0038x over previous
import functools

import jax
import jax.numpy as jnp
from jax import lax
from jax.experimental import pallas as pl
from jax.experimental.pallas import tpu as pltpu
from jax.experimental.pallas import tpu_sc as plsc

_NC = 2
_NS = 16
_L = 16
_NW = _NC * _NS

_B = 16384
_D = 32
_BPW = _B // _NW
_NB = 4


def _mf_body(u_hbm, v_hbm, ueT_hbm, veT_hbm, out_hbm,
             ui_v, vi_v, c32, du0, du1, dv0, dv1, o_v,
             semu0, semu1, semv0, semv1):
    wid = lax.axis_index("s") * _NC + lax.axis_index("c")
    base = wid * _BPW

    iota = lax.iota(jnp.int32, _L)
    c32[pl.ds(0, _L)] = iota
    c32[pl.ds(_L, _L)] = iota + _L

    pltpu.sync_copy(u_hbm.at[pl.ds(base, _BPW)], ui_v)
    pltpu.sync_copy(v_hbm.at[pl.ds(base, _BPW)], vi_v)

    dub = (du0, du1)
    dvb = (dv0, dv1)
    semub = (semu0, semu1)
    semvb = (semv0, semv1)

    def fire(u16, v16, b):
        p = b & 1
        cps = []
        for j in range(_NB):
            ru = u16[b * _NB + j]
            rv = v16[b * _NB + j]
            j0u = pl.multiple_of(ru & -128, 128)
            j0v = pl.multiple_of(rv & -128, 128)
            cps.append(pltpu.async_copy(
                ueT_hbm.at[c32, pl.ds(j0u, 128)], dub[p].at[j], semub[p]))
            cps.append(pltpu.async_copy(
                veT_hbm.at[c32, pl.ds(j0v, 128)], dvb[p].at[j], semvb[p]))
        return cps

    def wait_batch(p):
        for j in range(_NB):
            pltpu.make_async_copy(
                ueT_hbm.at[c32, pl.ds(0, 128)], dub[p].at[j], semub[p]
            ).wait()
            pltpu.make_async_copy(
                veT_hbm.at[c32, pl.ds(0, 128)], dvb[p].at[j], semvb[p]
            ).wait()

    def process(u16, v16, b, gbase):
        p = b & 1
        for j in range(_NB):
            ru = u16[b * _NB + j]
            rv = v16[b * _NB + j]
            rmu = jnp.zeros((_L,), jnp.int32) + (ru & 127)
            rmv = jnp.zeros((_L,), jnp.int32) + (rv & 127)
            du2 = dub[p].at[j]
            dv2 = dvb[p].at[j]
            a0 = plsc.load_gather(du2, [iota, rmu])
            a1 = plsc.load_gather(du2, [iota + _L, rmu])
            b0 = plsc.load_gather(dv2, [iota, rmv])
            b1 = plsc.load_gather(dv2, [iota + _L, rmv])
            e = a0 * b0 + a1 * b1
            dot = lax.reduce_sum_p.bind(e, axes=(0,))
            val = jnp.zeros((_L,), jnp.float32) + dot
            pos = jnp.zeros((_L,), jnp.int32) + (gbase + b * _NB + j)
            plsc.store_scatter(o_v, [pos], val, mask=iota == 0)

    _NBG = _L // _NB

    u16_0 = ui_v[pl.ds(0, _L)]
    v16_0 = vi_v[pl.ds(0, _L)]
    fire(u16_0, v16_0, 0)

    def group_body(g, carry):
        gbase = pl.multiple_of(g * _L, _L)
        u16 = ui_v[pl.ds(gbase, _L)]
        v16 = vi_v[pl.ds(gbase, _L)]
        nbase = pl.multiple_of((g + 1) * _L % _BPW, _L)
        u16n = ui_v[pl.ds(nbase, _L)]
        v16n = vi_v[pl.ds(nbase, _L)]

        for b in range(_NBG):
            if b + 1 < _NBG:
                fire(u16, v16, b + 1)
            else:
                fire(u16n, v16n, 0)
            wait_batch(b & 1)
            process(u16, v16, b, gbase)
        return carry

    lax.fori_loop(0, _BPW // _L, group_body, 0)
    wait_batch(0)

    def sig(i, carry):
        sl = pl.ds(i * _L, _L)
        o_v[sl] = 4.0 / (1.0 + jnp.exp(-o_v[sl])) + 1.0
        return carry

    lax.fori_loop(0, _BPW // _L, sig, 0)

    pltpu.sync_copy(o_v, out_hbm.at[pl.ds(base, _BPW)])


def kernel(u, v, user_emb, movie_emb):
    ueT = user_emb.T
    veT = movie_emb.T
    mesh = plsc.VectorSubcoreMesh(core_axis_name="c", subcore_axis_name="s")
    run = functools.partial(
        pl.kernel,
        out_type=jax.ShapeDtypeStruct((_B,), jnp.float32),
        mesh=mesh,
        compiler_params=pltpu.CompilerParams(
            needs_layout_passes=False, disable_bounds_checks=True
        ),
        scratch_types=[
            pltpu.VMEM((_BPW,), jnp.int32),
            pltpu.VMEM((_BPW,), jnp.int32),
            pltpu.VMEM((2 * _L,), jnp.int32),
            pltpu.VMEM((_NB, _D, 128), jnp.float32),
            pltpu.VMEM((_NB, _D, 128), jnp.float32),
            pltpu.VMEM((_NB, _D, 128), jnp.float32),
            pltpu.VMEM((_NB, _D, 128), jnp.float32),
            pltpu.VMEM((_BPW,), jnp.float32),
            pltpu.SemaphoreType.DMA,
            pltpu.SemaphoreType.DMA,
            pltpu.SemaphoreType.DMA,
            pltpu.SemaphoreType.DMA,
        ],
    )(_mf_body)
    return run(u, v, ueT, veT)
